# baseline (speedup 1.0000x reference)
import jax
import jax.numpy as jnp
from jax import lax
from jax.experimental import pallas as pl
from jax.experimental.pallas import tpu as pltpu
from jax.experimental.pallas import tpu_sc as plsc

_L = 17
_N = 384
_PLANES = 6
_NW = 32
_RPW = _N // _NW
_WIN = _RPW + 4
_TAB = 4920
_GROUPS = _N // 16
_OW = 2 * _N
_QBITS = 11

_OFFS = {'h': ((0, 1), (0, 2)), 'd': ((1, 1), (2, 2)),
         'b': ((1, 2), (2, 1)), 'v': ((1, 0), (2, 0))}


def _rot_disp(dy, dx, r):
    return [(dy, dx), (dx, -dy), (-dy, -dx), (-dx, dy)][r]


def _out_perm(u, v, r):
    return [(u, v), (v, 1 - u), (1 - u, 1 - v), (1 - v, u)][r]


def _idx_groups():
    groups = {}
    order = []
    for ki, k in enumerate(['h', 'd', 'b', 'v']):
        (o1, o2) = _OFFS[k]
        for r in range(4):
            d1 = _rot_disp(o1[0], o1[1], r)
            d2 = _rot_disp(o2[0], o2[1], r)
            perm = [0] * 4
            for u in (0, 1):
                for v in (0, 1):
                    up, vp = _out_perm(u, v, r)
                    perm[u * 2 + v] = up * 2 + vp
            key = (d1, d2)
            if key not in groups:
                groups[key] = []
                order.append(key)
            groups[key].append((ki, tuple(perm)))
    return [(d1, d2, groups[(d1, d2)]) for (d1, d2) in order]


_IDX_GROUPS = _idx_groups()
_PAIRS = []
for (_d1, _d2, _members) in _IDX_GROUPS:
    for (_ki, _perm) in _members:
        for pr in ((_perm[0], _perm[1]), (_perm[2], _perm[3])):
            if pr not in _PAIRS:
                _PAIRS.append(pr)

_DYS = (-2, -1, 0, 1, 2)
_DXS = (-2, -1, 0, 1, 2)


def _body(img_ref, tabs_ref, out_ref, *scratch):
    tab_refs = scratch[0:8]
    inbuf = scratch[8]
    outbuf = scratch[9]

    cid = lax.axis_index("c")
    sid = lax.axis_index("s")
    wid = sid * 2 + cid
    row0 = wid * _RPW
    ws = jnp.maximum(jnp.minimum(row0 - 2, _N - _WIN), 0)

    for i in range(8):
        pltpu.sync_copy(tabs_ref.at[i], tab_refs[i])

    iota = lax.iota(jnp.int32, 16)
    iota2 = iota * 2
    scale = jnp.float32(1.0 / (1 << _QBITS))

    for t in range(_PLANES):
        pltpu.sync_copy(img_ref.at[pl.ds(t * _N * _N + ws * _N, _WIN * _N)],
                        inbuf)

        def pair_body(p, carry):
            y0 = row0 + 2 * p
            rbs = []
            for j in range(6):
                yy = jnp.maximum(jnp.minimum(y0 - 2 + j, _N - 1), 0)
                rbs.append((yy - ws) * _N)

            def grp_body(g, c2):
                x = g * 16
                cvs = {}
                for dx in _DXS:
                    cvs[dx] = jnp.maximum(
                        jnp.minimum(iota + (x + dx), _N - 1), 0)
                loads = {}
                for j in range(6):
                    for dx in _DXS:
                        loads[(j, dx)] = plsc.load_gather(
                            inbuf, [cvs[dx] + rbs[j]])
                for r in (0, 1):
                    nb = {(dy, dx): loads[(dy + 2 + r, dx)]
                          for dy in _DYS for dx in _DXS}
                    a289 = nb[(0, 0)] * (_L * _L)
                    paccs = {pr: jnp.zeros((32,), jnp.int16)
                             for pr in _PAIRS}
                    for (d1, d2, members) in _IDX_GROUPS:
                        idx = a289 + nb[d1] * _L + nb[d2]
                        for (ki, perm) in members:
                            pk_t = plsc.load_gather(tab_refs[ki * 2], [idx])
                            pk_b = plsc.load_gather(tab_refs[ki * 2 + 1],
                                                    [idx])
                            pt = (perm[0], perm[1])
                            pb = (perm[2], perm[3])
                            paccs[pt] = paccs[pt] + plsc.bitcast(
                                pk_t, jnp.int16)
                            paccs[pb] = paccs[pb] + plsc.bitcast(
                                pk_b, jnp.int16)
                    slot = [None] * 4
                    for pr in _PAIRS:
                        a32 = plsc.bitcast(paccs[pr], jnp.int32)
                        lo = (a32 << 16) >> 16
                        hi = a32 >> 16
                        for (s, v) in ((pr[0], lo), (pr[1], hi)):
                            slot[s] = v if slot[s] is None else slot[s] + v
                    stb = iota2 + ((2 * p + r) * (2 * _OW) + x * 2)
                    offs = (0, 1, _OW, _OW + 1)
                    for s in range(4):
                        plsc.store_scatter(
                            outbuf, [stb + offs[s]],
                            slot[s].astype(jnp.float32) * scale)
                return c2

            lax.fori_loop(0, _GROUPS, grp_body, 0)
            return carry

        lax.fori_loop(0, _RPW // 2, pair_body, 0)
        pltpu.sync_copy(
            outbuf,
            out_ref.at[pl.ds(t * _OW * _OW + row0 * (2 * _OW),
                             _RPW * 2 * _OW)])


def kernel(img_lr, h_weight, d_weight, b_weight, v_weight):
    img = img_lr.astype(jnp.int32).reshape(_PLANES * _N * _N)

    rows = []
    for w in (h_weight, d_weight, b_weight, v_weight):
        wf = (w * (0.25 * (1 << _QBITS))).reshape(_L ** 3, 4)
        q = jnp.clip(jnp.round(wf), -32768, 32767).astype(jnp.int32)
        qm = q & 0xFFFF
        top = (q[:, 1] << 16) | qm[:, 0]
        bot = (q[:, 3] << 16) | qm[:, 2]
        rows.append(jnp.pad(top, (0, _TAB - _L ** 3)))
        rows.append(jnp.pad(bot, (0, _TAB - _L ** 3)))
    tabs = jnp.stack(rows)

    mesh = plsc.VectorSubcoreMesh(core_axis_name="c", subcore_axis_name="s")
    scratch = [pltpu.VMEM((_TAB,), jnp.int32) for _ in range(8)]
    scratch.append(pltpu.VMEM((_WIN * _N,), jnp.int32))
    scratch.append(pltpu.VMEM((_RPW * 2 * _OW,), jnp.float32))

    out = pl.kernel(
        _body,
        out_type=jax.ShapeDtypeStruct((_PLANES * _OW * _OW,), jnp.float32),
        mesh=mesh,
        scratch_types=scratch,
        compiler_params=pltpu.CompilerParams(needs_layout_passes=False),
    )(img, tabs)
    return out.reshape(2, 3, _OW, _OW)


# speedup vs baseline: 1.0749x; 1.0749x over previous
import jax
import jax.numpy as jnp
from jax import lax
from jax.experimental import pallas as pl
from jax.experimental.pallas import tpu as pltpu
from jax.experimental.pallas import tpu_sc as plsc

_L = 17
_N = 384
_PLANES = 6
_NW = 32
_RPW = _N // _NW
_WIN = _RPW + 4
_TAB = 4920
_GROUPS = _N // 16
_OW = 2 * _N
_PW = 392

_OFFS = {'h': ((0, 1), (0, 2)), 'd': ((1, 1), (2, 2)),
         'b': ((1, 2), (2, 1)), 'v': ((1, 0), (2, 0))}


def _rot_disp(dy, dx, r):
    return [(dy, dx), (dx, -dy), (-dy, -dx), (-dx, dy)][r]


def _out_perm(u, v, r):
    return [(u, v), (v, 1 - u), (1 - u, 1 - v), (1 - v, u)][r]


def _idx_groups():
    groups = {}
    order = []
    for ki, k in enumerate(['h', 'd', 'b', 'v']):
        (o1, o2) = _OFFS[k]
        for r in range(4):
            d1 = _rot_disp(o1[0], o1[1], r)
            d2 = _rot_disp(o2[0], o2[1], r)
            perm = [0] * 4
            for u in (0, 1):
                for v in (0, 1):
                    up, vp = _out_perm(u, v, r)
                    perm[u * 2 + v] = up * 2 + vp
            key = (d1, d2)
            if key not in groups:
                groups[key] = []
                order.append(key)
            groups[key].append((ki, tuple(perm)))
    return [(d1, d2, groups[(d1, d2)]) for (d1, d2) in order]


_IDX_GROUPS = _idx_groups()
_DYS = (-2, -1, 0, 1, 2)


def _body(img_ref, tabs_ref, out_ref, *scratch):
    tab_refs = scratch[0:8]
    inbufs = scratch[8:10]
    outbufs = scratch[10:12]
    sem_tab = scratch[12]
    sem_in = scratch[13:15]
    sem_out = scratch[15:17]

    cid = lax.axis_index("c")
    sid = lax.axis_index("s")
    wid = sid * 2 + cid
    row0 = wid * _RPW
    ws = jnp.maximum(jnp.minimum(row0 - 2, _N - _WIN), 0)

    tab_copies = [pltpu.async_copy(tabs_ref.at[i], tab_refs[i], sem_tab)
                  for i in range(8)]

    def in_copy(t):
        return pltpu.async_copy(
            img_ref.at[pl.ds(t * _N * _PW + ws * _PW, _WIN * _PW)],
            inbufs[t % 2], sem_in[t % 2])

    in_handles = {0: in_copy(0)}
    out_handles = {}

    for c in tab_copies:
        c.wait()

    iota = lax.iota(jnp.int32, 16)
    iota2 = iota * 2

    for t in range(_PLANES):
        in_handles[t].wait()
        if t + 1 < _PLANES:
            in_handles[t + 1] = in_copy(t + 1)
        if t >= 2:
            out_handles[t - 2].wait()
        inbuf = inbufs[t % 2]
        outbuf = outbufs[t % 2]

        def pair_body(p, carry):
            y0 = row0 + 2 * p
            rbs = []
            for j in range(6):
                yy = jnp.maximum(jnp.minimum(y0 - 2 + j, _N - 1), 0)
                rbs.append((yy - ws) * _PW)

            def grp_body(g, c2):
                x = g * 16
                cvs = {dxb: iota + (x + dxb + 2) for dxb in (-2, 0, 2)}
                val = {}
                for j in range(6):
                    w_m2 = plsc.load_gather(inbuf, [cvs[-2] + rbs[j]])
                    w_0 = plsc.load_gather(inbuf, [cvs[0] + rbs[j]])
                    w_p2 = plsc.load_gather(inbuf, [cvs[2] + rbs[j]])
                    val[(j, -2)] = w_m2 & 0xFFFF
                    val[(j, -1)] = lax.shift_right_logical(w_m2, 16)
                    val[(j, 0)] = w_0 & 0xFFFF
                    val[(j, 1)] = lax.shift_right_logical(w_0, 16)
                    val[(j, 2)] = w_p2 & 0xFFFF
                for r in (0, 1):
                    nb = {(dy, dx): val[(dy + 2 + r, dx)]
                          for dy in _DYS for dx in _DYS}
                    a289 = nb[(0, 0)] * (_L * _L)
                    accs = [jnp.zeros((16,), jnp.float32) for _ in range(4)]
                    for (d1, d2, members) in _IDX_GROUPS:
                        idx = a289 + nb[d1] * _L + nb[d2]
                        for (ki, perm) in members:
                            pk_t = plsc.load_gather(tab_refs[ki * 2], [idx])
                            pk_b = plsc.load_gather(tab_refs[ki * 2 + 1],
                                                    [idx])
                            w00 = plsc.bitcast(lax.shift_left(pk_t, 16),
                                               jnp.float32)
                            w01 = plsc.bitcast(pk_t, jnp.float32)
                            w10 = plsc.bitcast(lax.shift_left(pk_b, 16),
                                               jnp.float32)
                            w11 = plsc.bitcast(pk_b, jnp.float32)
                            accs[perm[0]] = accs[perm[0]] + w00
                            accs[perm[1]] = accs[perm[1]] + w01
                            accs[perm[2]] = accs[perm[2]] + w10
                            accs[perm[3]] = accs[perm[3]] + w11
                    stb = iota2 + ((2 * p + r) * (2 * _OW) + x * 2)
                    plsc.store_scatter(outbuf, [stb], accs[0])
                    plsc.store_scatter(outbuf, [stb + 1], accs[1])
                    plsc.store_scatter(outbuf, [stb + _OW], accs[2])
                    plsc.store_scatter(outbuf, [stb + _OW + 1], accs[3])
                return c2

            lax.fori_loop(0, _GROUPS, grp_body, 0)
            return carry

        lax.fori_loop(0, _RPW // 2, pair_body, 0)
        out_handles[t] = pltpu.async_copy(
            outbuf,
            out_ref.at[pl.ds(t * _OW * _OW + row0 * (2 * _OW),
                             _RPW * 2 * _OW)],
            sem_out[t % 2])

    out_handles[_PLANES - 2].wait()
    out_handles[_PLANES - 1].wait()


def kernel(img_lr, h_weight, d_weight, b_weight, v_weight):
    img_i = img_lr.astype(jnp.int32).reshape(_PLANES, _N, _N)
    pp = jnp.pad(img_i, ((0, 0), (0, 0), (2, 2)), mode='edge')
    nxt = jnp.concatenate([pp[:, :, 1:], pp[:, :, -1:]], axis=2)
    packed = jnp.pad(pp | (nxt << 16), ((0, 0), (0, 0), (0, _PW - 388)))
    img = packed.reshape(_PLANES * _N * _PW)

    rows = []
    for w in (h_weight, d_weight, b_weight, v_weight):
        wf = (w * 0.25).reshape(_L ** 3, 4)
        bits = lax.bitcast_convert_type(
            wf.astype(jnp.bfloat16), jnp.uint16).astype(jnp.uint32)
        top = lax.bitcast_convert_type(
            (bits[:, 1] << 16) | bits[:, 0], jnp.int32)
        bot = lax.bitcast_convert_type(
            (bits[:, 3] << 16) | bits[:, 2], jnp.int32)
        rows.append(jnp.pad(top, (0, _TAB - _L ** 3)))
        rows.append(jnp.pad(bot, (0, _TAB - _L ** 3)))
    tabs = jnp.stack(rows)

    mesh = plsc.VectorSubcoreMesh(core_axis_name="c", subcore_axis_name="s")
    scratch = [pltpu.VMEM((_TAB,), jnp.int32) for _ in range(8)]
    scratch += [pltpu.VMEM((_WIN * _PW,), jnp.int32) for _ in range(2)]
    scratch += [pltpu.VMEM((_RPW * 2 * _OW,), jnp.float32) for _ in range(2)]
    scratch += [pltpu.SemaphoreType.DMA for _ in range(5)]

    out = pl.kernel(
        _body,
        out_type=jax.ShapeDtypeStruct((_PLANES * _OW * _OW,), jnp.float32),
        mesh=mesh,
        scratch_types=scratch,
        compiler_params=pltpu.CompilerParams(needs_layout_passes=False),
    )(img, tabs)
    return out.reshape(2, 3, _OW, _OW)


# speedup vs baseline: 1.2361x; 1.1499x over previous
import jax
import jax.numpy as jnp
from jax import lax
from jax.experimental import pallas as pl
from jax.experimental.pallas import tpu as pltpu
from jax.experimental.pallas import tpu_sc as plsc

_L = 17
_N = 384
_PLANES = 6
_NW = 32
_RPW = _N // _NW
_WIN = _RPW + 4
_TAB = 4920
_GROUPS = _N // 16
_OW = 2 * _N
_PW = 392

_OFFS = {'h': ((0, 1), (0, 2)), 'd': ((1, 1), (2, 2)),
         'b': ((1, 2), (2, 1)), 'v': ((1, 0), (2, 0))}


def _rot_disp(dy, dx, r):
    return [(dy, dx), (dx, -dy), (-dy, -dx), (-dx, dy)][r]


def _out_perm(u, v, r):
    return [(u, v), (v, 1 - u), (1 - u, 1 - v), (1 - v, u)][r]


def _idx_groups():
    groups = {}
    order = []
    for ki, k in enumerate(['h', 'd', 'b', 'v']):
        (o1, o2) = _OFFS[k]
        for r in range(4):
            d1 = _rot_disp(o1[0], o1[1], r)
            d2 = _rot_disp(o2[0], o2[1], r)
            perm = [0] * 4
            for u in (0, 1):
                for v in (0, 1):
                    up, vp = _out_perm(u, v, r)
                    perm[u * 2 + v] = up * 2 + vp
            key = (d1, d2)
            if key not in groups:
                groups[key] = []
                order.append(key)
            groups[key].append((ki, tuple(perm)))
    return [(d1, d2, groups[(d1, d2)]) for (d1, d2) in order]


_IDX_GROUPS = _idx_groups()
_DYS = (-2, -1, 0, 1, 2)

_TAB_PLAN = []
_SHARED_COL = {}
_NUM_PAIRS = 0
for (_d1, _d2, _members) in _IDX_GROUPS:
    if len(_members) > 1:
        _TAB_PLAN.append((_NUM_PAIRS, None))
        _NUM_PAIRS += 1
    else:
        (_ki, _perm) = _members[0]
        if _ki not in _SHARED_COL:
            _SHARED_COL[_ki] = _NUM_PAIRS
            _NUM_PAIRS += 1
        _TAB_PLAN.append((_SHARED_COL[_ki], _perm))


def _body(img_ref, tabs_ref, out_ref, *scratch):
    tab_refs = scratch[0:2 * _NUM_PAIRS]
    inbufs = scratch[2 * _NUM_PAIRS:2 * _NUM_PAIRS + 2]
    outbufs = scratch[2 * _NUM_PAIRS + 2:2 * _NUM_PAIRS + 4]
    sem_tab = scratch[2 * _NUM_PAIRS + 4]
    sem_in = scratch[2 * _NUM_PAIRS + 5:2 * _NUM_PAIRS + 7]
    sem_out = scratch[2 * _NUM_PAIRS + 7:2 * _NUM_PAIRS + 9]

    cid = lax.axis_index("c")
    sid = lax.axis_index("s")
    wid = sid * 2 + cid
    row0 = wid * _RPW
    ws = jnp.maximum(jnp.minimum(row0 - 2, _N - _WIN), 0)

    tab_copies = [
        pltpu.async_copy(tabs_ref.at[i], tab_refs[i], sem_tab)
        for i in range(2 * _NUM_PAIRS)]

    def in_copy(t):
        return pltpu.async_copy(
            img_ref.at[pl.ds(t * _N * _PW + ws * _PW, _WIN * _PW)],
            inbufs[t % 2], sem_in[t % 2])

    in_handles = {0: in_copy(0)}
    out_handles = {}

    for c in tab_copies:
        c.wait()

    iota = lax.iota(jnp.int32, 16)
    iota2 = iota * 2

    for t in range(_PLANES):
        in_handles[t].wait()
        if t + 1 < _PLANES:
            in_handles[t + 1] = in_copy(t + 1)
        if t >= 2:
            out_handles[t - 2].wait()
        inbuf = inbufs[t % 2]
        outbuf = outbufs[t % 2]

        def pair_body(p, carry):
            y0 = row0 + 2 * p
            rbs = []
            for j in range(6):
                yy = jnp.maximum(jnp.minimum(y0 - 2 + j, _N - 1), 0)
                rbs.append((yy - ws) * _PW)

            def grp_body(g, c2):
                x = g * 16
                cvs = {dxb: iota + (x + dxb + 2) for dxb in (-2, 0, 2)}
                val = {}
                for j in range(6):
                    w_m2 = plsc.load_gather(inbuf, [cvs[-2] + rbs[j]])
                    w_0 = plsc.load_gather(inbuf, [cvs[0] + rbs[j]])
                    w_p2 = plsc.load_gather(inbuf, [cvs[2] + rbs[j]])
                    val[(j, -2)] = w_m2 & 0xFFFF
                    val[(j, -1)] = lax.shift_right_logical(w_m2, 16)
                    val[(j, 0)] = w_0 & 0xFFFF
                    val[(j, 1)] = lax.shift_right_logical(w_0, 16)
                    val[(j, 2)] = w_p2 & 0xFFFF
                for r in (0, 1):
                    nb = {(dy, dx): val[(dy + 2 + r, dx)]
                          for dy in _DYS for dx in _DYS}
                    a289 = nb[(0, 0)] * (_L * _L)
                    accs = [None] * 4
                    for gi, (d1, d2, members) in enumerate(_IDX_GROUPS):
                        (pair_i, perm) = _TAB_PLAN[gi]
                        if perm is None:
                            perm = (0, 1, 2, 3)
                        idx = a289 + nb[d1] * _L + nb[d2]
                        pk_t = plsc.load_gather(tab_refs[2 * pair_i],
                                                [idx])
                        pk_b = plsc.load_gather(tab_refs[2 * pair_i + 1],
                                                [idx])
                        w = (plsc.bitcast(lax.shift_left(pk_t, 16),
                                          jnp.float32),
                             plsc.bitcast(pk_t, jnp.float32),
                             plsc.bitcast(lax.shift_left(pk_b, 16),
                                          jnp.float32),
                             plsc.bitcast(pk_b, jnp.float32))
                        for uv in range(4):
                            s = perm[uv]
                            accs[s] = (w[uv] if accs[s] is None
                                       else accs[s] + w[uv])
                    stb = iota2 + ((2 * p + r) * (2 * _OW) + x * 2)
                    plsc.store_scatter(outbuf, [stb], accs[0])
                    plsc.store_scatter(outbuf, [stb + 1], accs[1])
                    plsc.store_scatter(outbuf, [stb + _OW], accs[2])
                    plsc.store_scatter(outbuf, [stb + _OW + 1], accs[3])
                return c2

            lax.fori_loop(0, _GROUPS, grp_body, 0)
            return carry

        lax.fori_loop(0, _RPW // 2, pair_body, 0)
        out_handles[t] = pltpu.async_copy(
            outbuf,
            out_ref.at[pl.ds(t * _OW * _OW + row0 * (2 * _OW),
                             _RPW * 2 * _OW)],
            sem_out[t % 2])

    out_handles[_PLANES - 2].wait()
    out_handles[_PLANES - 1].wait()


def _pack_cols(c0, c1, c2, c3):
    cols = []
    for lo, hi in ((c0, c1), (c2, c3)):
        b_lo = lax.bitcast_convert_type(
            lo.astype(jnp.bfloat16), jnp.uint16).astype(jnp.uint32)
        b_hi = lax.bitcast_convert_type(
            hi.astype(jnp.bfloat16), jnp.uint16).astype(jnp.uint32)
        w = lax.bitcast_convert_type((b_hi << 16) | b_lo, jnp.int32)
        cols.append(jnp.pad(w, (0, _TAB - _L ** 3)))
    return cols


def kernel(img_lr, h_weight, d_weight, b_weight, v_weight):
    img_i = img_lr.astype(jnp.int32).reshape(_PLANES, _N, _N)
    pp = jnp.pad(img_i, ((0, 0), (0, 0), (2, 2)), mode='edge')
    nxt = jnp.concatenate([pp[:, :, 1:], pp[:, :, -1:]], axis=2)
    packed = jnp.pad(pp | (nxt << 16), ((0, 0), (0, 0), (0, _PW - 388)))
    img = packed.reshape(_PLANES * _N * _PW)

    wk = [(w * 0.25).reshape(_L ** 3, 4)
          for w in (h_weight, d_weight, b_weight, v_weight)]
    cols = [None] * (2 * _NUM_PAIRS)
    for gi, (d1, d2, members) in enumerate(_IDX_GROUPS):
        (pair_i, _) = _TAB_PLAN[gi]
        if len(members) > 1:
            slot_cols = []
            for s in range(4):
                acc = None
                for (ki, perm) in members:
                    uv = perm.index(s)
                    c = wk[ki][:, uv]
                    acc = c if acc is None else acc + c
                slot_cols.append(acc)
            pc = _pack_cols(*slot_cols)
        else:
            (ki, _) = members[0]
            pc = _pack_cols(wk[ki][:, 0], wk[ki][:, 1],
                            wk[ki][:, 2], wk[ki][:, 3])
        cols[2 * pair_i] = pc[0]
        cols[2 * pair_i + 1] = pc[1]
    tabs = jnp.stack(cols)

    mesh = plsc.VectorSubcoreMesh(core_axis_name="c", subcore_axis_name="s")
    scratch = [pltpu.VMEM((_TAB,), jnp.int32)
               for _ in range(2 * _NUM_PAIRS)]
    scratch += [pltpu.VMEM((_WIN * _PW,), jnp.int32) for _ in range(2)]
    scratch += [pltpu.VMEM((_RPW * 2 * _OW,), jnp.float32)
                for _ in range(2)]
    scratch += [pltpu.SemaphoreType.DMA for _ in range(5)]

    out = pl.kernel(
        _body,
        out_type=jax.ShapeDtypeStruct((_PLANES * _OW * _OW,), jnp.float32),
        mesh=mesh,
        scratch_types=scratch,
        compiler_params=pltpu.CompilerParams(needs_layout_passes=False),
    )(img, tabs)
    return out.reshape(2, 3, _OW, _OW)
